# baseline (speedup 1.0000x reference)
import functools

import jax
import jax.numpy as jnp
from jax import lax
from jax.experimental import pallas as pl
from jax.experimental.pallas import tpu as pltpu
from jax.experimental.pallas import tpu_sc as plsc

_CHUNK = 16
_NBUF = 4


@functools.cache
def _sc_gather_call(n_rows: int, n_cols: int, chunk: int, nbuf: int):
    info = plsc.get_sparse_core_info()
    nw = info.num_cores * info.num_subcores
    rows_per_worker = n_rows // nw
    n_chunks = rows_per_worker // chunk
    m = n_chunks // 2
    mesh = plsc.VectorSubcoreMesh(core_axis_name="c", subcore_axis_name="s")

    @functools.partial(
        pl.kernel,
        mesh=mesh,
        out_type=jax.ShapeDtypeStruct((n_rows, n_cols), jnp.float32),
        scratch_types=[
            pltpu.VMEM((rows_per_worker,), jnp.int32),
            pltpu.VMEM((nbuf, chunk, n_cols), jnp.float32),
            pltpu.VMEM_SHARED(
                (info.num_subcores, 2, chunk, n_cols), jnp.float32
            ),
        ]
        + [pltpu.SemaphoreType.DMA] * (2 * nbuf + 4),
    )
    def gather(x_hbm, idx_hbm, out_hbm, idx_v, rows_a, rows_b, *sems):
        gsem_a = sems[:nbuf]
        ssem_a = sems[nbuf : 2 * nbuf]
        gsem_b = sems[2 * nbuf : 2 * nbuf + 2]
        ssem_b = sems[2 * nbuf + 2 :]
        sid = lax.axis_index("s")
        wid = sid * info.num_cores + lax.axis_index("c")
        base = wid * rows_per_worker
        pltpu.sync_copy(idx_hbm.at[pl.ds(base, rows_per_worker)], idx_v)

        def start_gather_a(i, b):
            idx_slice = idx_v.at[pl.ds(2 * i * chunk, chunk)]
            pltpu.async_copy(x_hbm.at[idx_slice], rows_a.at[b], gsem_a[b])

        def wait_gather_a(i, b):
            idx_slice = idx_v.at[pl.ds(2 * i * chunk, chunk)]
            pltpu.make_async_copy(
                x_hbm.at[idx_slice], rows_a.at[b], gsem_a[b]
            ).wait()

        def start_scatter_a(i, b):
            pltpu.async_copy(
                rows_a.at[b],
                out_hbm.at[pl.ds(base + 2 * i * chunk, chunk)],
                ssem_a[b],
            )

        def wait_scatter_a(i, b):
            pltpu.make_async_copy(
                rows_a.at[b],
                out_hbm.at[pl.ds(base + 2 * i * chunk, chunk)],
                ssem_a[b],
            ).wait()

        def start_gather_b(i, b):
            vec = idx_v[pl.ds((2 * i + 1) * chunk, chunk)]
            for k in range(chunk):
                pltpu.async_copy(
                    x_hbm.at[pl.ds(vec[k], 1)],
                    rows_b.at[sid, b, pl.ds(k, 1)],
                    gsem_b[b],
                )

        def wait_gather_b(i, b):
            pltpu.make_async_copy(
                x_hbm.at[pl.ds(0, chunk)], rows_b.at[sid, b], gsem_b[b]
            ).wait()

        def start_scatter_b(i, b):
            pltpu.async_copy(
                rows_b.at[sid, b],
                out_hbm.at[pl.ds(base + (2 * i + 1) * chunk, chunk)],
                ssem_b[b],
            )

        def wait_scatter_b(i, b):
            pltpu.make_async_copy(
                rows_b.at[sid, b],
                out_hbm.at[pl.ds(base + (2 * i + 1) * chunk, chunk)],
                ssem_b[b],
            ).wait()

        for b in range(nbuf):
            start_gather_a(b, b)
        for b in range(2):
            start_gather_b(b, b)

        def body(p, carry):
            for j in range(nbuf):
                i = p * nbuf + j

                wait_gather_a(i, j)
                start_scatter_a(i, j)

                @pl.when(jnp.logical_and(i >= 2, i <= m - nbuf + 1))
                def _():
                    b1 = (j - 2) % nbuf
                    wait_scatter_a(i - 2, b1)
                    start_gather_a(i + nbuf - 2, b1)

                jb = j % 2
                wait_gather_b(i, jb)
                start_scatter_b(i, jb)

                @pl.when(i + 2 < m)
                def _():
                    wait_scatter_b(i, jb)
                    start_gather_b(i + 2, jb)

            return carry

        lax.fori_loop(0, m // nbuf, body, 0)
        for t in range(nbuf):
            i = m - nbuf + t
            wait_scatter_a(i, i % nbuf)
        for t in range(2):
            i = m - 2 + t
            wait_scatter_b(i, i % 2)

    return gather


@functools.cache
def _perm_constants(B: int, L: int):
    import numpy as np

    with jax.ensure_compile_time_eval():
        perm1d = np.asarray(jax.random.permutation(jax.random.key(42), L))
    perm = np.tile(perm1d[None, :], (B, 1))
    src = (
        np.arange(B, dtype=np.int32)[:, None] * L + perm1d[None, :]
    ).reshape(-1)
    return jnp.asarray(perm), jnp.asarray(src.astype(np.int32))


def kernel(x):
    B, L, C = x.shape
    perm, src = _perm_constants(B, L)
    out = _sc_gather_call(B * L, C, _CHUNK, _NBUF)(x.reshape(B * L, C), src)
    return out.reshape(B, L, C), perm
